# Initial kernel scaffold; baseline (speedup 1.0000x reference)
import jax
import jax.numpy as jnp
from jax.experimental import pallas as pl


def kernel(x, y, stds, covar_factors):
    raise NotImplementedError("write your pallas kernel here")


# speedup vs baseline: 1.3799x; 1.3799x over previous
import jax
import jax.numpy as jnp
from jax import lax
from jax.experimental import pallas as pl
from jax.experimental.pallas import tpu as pltpu
from jax.experimental.pallas import tpu_sc as plsc

NC = 2
NS = 16
NW = NC * NS
L = 16


def _body(xt_hbm, yt_hbm, covf_hbm, stds_hbm, out_hbm,
          xv_ref, yv_ref, tab_ref, sv_ref, acc_ref):
    nf, nb_cat, rank = covf_hbm.shape
    bw = acc_ref.shape[0]
    ng = bw // L
    wid = lax.axis_index("s") * NC + lax.axis_index("c")
    base = wid * bw

    def zero_body(g, _):
        acc_ref[pl.ds(g * L, L)] = jnp.zeros((L,), jnp.float32)
        return 0
    lax.fori_loop(0, ng, zero_body, 0)

    def field_body(f, _):
        pltpu.sync_copy(covf_hbm.at[f], tab_ref)
        pltpu.sync_copy(stds_hbm.at[f], sv_ref)
        pltpu.sync_copy(xt_hbm.at[f, pl.ds(base, bw)], xv_ref)
        pltpu.sync_copy(yt_hbm.at[f, pl.ds(base, bw)], yv_ref)

        def group_body(g, _):
            xv = xv_ref[pl.ds(g * L, L)]
            yv = yv_ref[pl.ds(g * L, L)]
            s = plsc.load_gather(sv_ref, [xv])
            acc = jnp.where(xv == yv, s * s, jnp.zeros((L,), jnp.float32))
            for r in range(rank):
                cr = jnp.full((L,), r, jnp.int32)
                xa = plsc.load_gather(tab_ref, [xv, cr])
                ya = plsc.load_gather(tab_ref, [yv, cr])
                acc = acc + xa * ya
            acc_ref[pl.ds(g * L, L)] = acc_ref[pl.ds(g * L, L)] + acc
            return 0
        lax.fori_loop(0, ng, group_body, 0)
        return 0
    lax.fori_loop(0, nf, field_body, 0)

    pltpu.sync_copy(acc_ref, out_hbm.at[pl.ds(base, bw)])


def kernel(x, y, stds, covar_factors):
    b, nf = x.shape
    nb_cat, rank = covar_factors.shape[1], covar_factors.shape[2]
    bw = b // NW
    xt = x.T
    yt = y.T
    mesh = plsc.VectorSubcoreMesh(
        core_axis_name="c", subcore_axis_name="s", num_cores=NC, num_subcores=NS)
    run = pl.kernel(
        _body,
        out_type=jax.ShapeDtypeStruct((b,), jnp.float32),
        mesh=mesh,
        scratch_types=[
            pltpu.VMEM((bw,), jnp.int32),
            pltpu.VMEM((bw,), jnp.int32),
            pltpu.VMEM((nb_cat, rank), jnp.float32),
            pltpu.VMEM((nb_cat,), jnp.float32),
            pltpu.VMEM((bw,), jnp.float32),
        ],
        compiler_params=pltpu.CompilerParams(needs_layout_passes=False),
    )
    return run(xt, yt, covar_factors, stds)


# speedup vs baseline: 3.9457x; 2.8594x over previous
import jax
import jax.numpy as jnp
from jax import lax
from jax.experimental import pallas as pl
from jax.experimental.pallas import tpu as pltpu
from jax.experimental.pallas import tpu_sc as plsc

NC = 2
NS = 16
NW = NC * NS
L = 16


def _body(xf_hbm, yf_hbm, cf_hbm, sf_hbm, out_hbm,
          xall_ref, yall_ref, tab0_ref, tab1_ref, sall_ref, acc_ref, sem0, sem1):
    nf, tabw = cf_hbm.shape
    nb_cat = sf_hbm.shape[0] // nf
    rank = tabw // nb_cat
    bw = acc_ref.shape[0]
    ng = bw // L
    wid = lax.axis_index("s") * NC + lax.axis_index("c")
    base = wid * bw

    cpx = pltpu.async_copy(xf_hbm.at[pl.ds(base * nf, bw * nf)], xall_ref, sem1)
    cpy = pltpu.async_copy(yf_hbm.at[pl.ds(base * nf, bw * nf)], yall_ref, sem1)
    cps = pltpu.async_copy(sf_hbm, sall_ref, sem1)
    pltpu.async_copy(cf_hbm.at[0], tab0_ref, sem0)
    cpx.wait()
    cpy.wait()
    cps.wait()

    def zero_body(g, _):
        acc_ref[pl.ds(g * L, L)] = jnp.zeros((L,), jnp.float32)
        return 0
    lax.fori_loop(0, ng, zero_body, 0)

    iota = lax.broadcasted_iota(jnp.int32, (L,), 0)
    iota_nf = iota * nf

    def compute_field(f, tab):
        fb = f * nb_cat

        def group_body(g, _):
            gb = g * (L * nf) + f
            ivx = iota_nf + gb
            xv = plsc.load_gather(xall_ref, [ivx])
            yv = plsc.load_gather(yall_ref, [ivx])
            s = plsc.load_gather(sall_ref, [xv + fb])
            acc = jnp.where(xv == yv, s * s, jnp.zeros((L,), jnp.float32))
            xb = xv * rank
            yb = yv * rank
            for r in range(rank):
                xa = plsc.load_gather(tab, [xb + r])
                ya = plsc.load_gather(tab, [yb + r])
                acc = acc + xa * ya
            sl = pl.ds(g * L, L)
            acc_ref[sl] = acc_ref[sl] + acc
            return 0
        lax.fori_loop(0, ng, group_body, 0)

    def pair_body(j, _):
        f0 = 2 * j
        f1 = f0 + 1
        f2 = jnp.minimum(f0 + 2, nf - 1)
        pltpu.async_copy(cf_hbm.at[f1], tab1_ref, sem1)
        pltpu.make_async_copy(cf_hbm.at[f0], tab0_ref, sem0).wait()
        compute_field(f0, tab0_ref)
        pltpu.async_copy(cf_hbm.at[f2], tab0_ref, sem0)
        pltpu.make_async_copy(cf_hbm.at[f1], tab1_ref, sem1).wait()
        compute_field(f1, tab1_ref)
        return 0
    lax.fori_loop(0, nf // 2, pair_body, 0)

    pltpu.make_async_copy(cf_hbm.at[nf - 1], tab0_ref, sem0).wait()

    pltpu.sync_copy(acc_ref, out_hbm.at[pl.ds(base, bw)])


def kernel(x, y, stds, covar_factors):
    b, nf = x.shape
    nb_cat, rank = covar_factors.shape[1], covar_factors.shape[2]
    bw = b // NW
    xf = x.reshape(-1)
    yf = y.reshape(-1)
    cf = covar_factors.reshape(nf, nb_cat * rank)
    sf = stds.reshape(-1)
    mesh = plsc.VectorSubcoreMesh(
        core_axis_name="c", subcore_axis_name="s", num_cores=NC, num_subcores=NS)
    run = pl.kernel(
        _body,
        out_type=jax.ShapeDtypeStruct((b,), jnp.float32),
        mesh=mesh,
        scratch_types=[
            pltpu.VMEM((bw * nf,), jnp.int32),
            pltpu.VMEM((bw * nf,), jnp.int32),
            pltpu.VMEM((nb_cat * rank,), jnp.float32),
            pltpu.VMEM((nb_cat * rank,), jnp.float32),
            pltpu.VMEM((nf * nb_cat,), jnp.float32),
            pltpu.VMEM((bw,), jnp.float32),
            pltpu.SemaphoreType.DMA,
            pltpu.SemaphoreType.DMA,
        ],
        compiler_params=pltpu.CompilerParams(needs_layout_passes=False),
    )
    return run(xf, yf, cf, sf)


# speedup vs baseline: 4.5057x; 1.1419x over previous
import jax
import jax.numpy as jnp
from jax import lax
from jax.experimental import pallas as pl
from jax.experimental.pallas import tpu as pltpu
from jax.experimental.pallas import tpu_sc as plsc

NC = 2
NS = 16
NW = NC * NS
L = 16


def _body(xf_hbm, yf_hbm, cf_hbm, sf_hbm, out_hbm,
          xall_ref, yall_ref, tab0_ref, tab1_ref, sall_ref, acc_ref, sem0, sem1):
    nf, tabw = cf_hbm.shape
    nb_cat = sf_hbm.shape[0] // nf
    rank = tabw // nb_cat
    bw = acc_ref.shape[0]
    ng = bw // L
    wid = lax.axis_index("s") * NC + lax.axis_index("c")
    base = wid * bw

    cpx = pltpu.async_copy(xf_hbm.at[pl.ds(base * nf, bw * nf)], xall_ref, sem1)
    cpy = pltpu.async_copy(yf_hbm.at[pl.ds(base * nf, bw * nf)], yall_ref, sem1)
    cps = pltpu.async_copy(sf_hbm, sall_ref, sem1)
    pltpu.async_copy(cf_hbm.at[0], tab0_ref, sem0)
    cpx.wait()
    cpy.wait()
    cps.wait()

    def zero_body(g, _):
        acc_ref[pl.ds(g * L, L)] = jnp.zeros((L,), jnp.float32)
        return 0
    lax.fori_loop(0, ng, zero_body, 0)

    iota = lax.broadcasted_iota(jnp.int32, (L,), 0)
    iota_nf = iota * nf

    def compute_field(f, tab):
        fb = f * nb_cat

        @plsc.parallel_loop(0, ng, unroll=2)
        def group_body(g):
            gb = g * (L * nf) + f
            ivx = iota_nf + gb
            xv = plsc.load_gather(xall_ref, [ivx])
            yv = plsc.load_gather(yall_ref, [ivx])
            s = plsc.load_gather(sall_ref, [xv + fb])
            xb = xv * rank
            yb = yv * rank
            na = 4
            accs = [jnp.zeros((L,), jnp.float32) for _ in range(na)]
            accs[na - 1] = jnp.where(
                xv == yv, s * s, jnp.zeros((L,), jnp.float32))
            for r in range(rank):
                xa = plsc.load_gather(tab, [xb + r])
                ya = plsc.load_gather(tab, [yb + r])
                accs[r % na] = accs[r % na] + xa * ya
            acc = (accs[0] + accs[1]) + (accs[2] + accs[3])
            sl = pl.ds(g * L, L)
            acc_ref[sl] = acc_ref[sl] + acc

    def pair_body(j, _):
        f0 = 2 * j
        f1 = f0 + 1
        f2 = jnp.minimum(f0 + 2, nf - 1)
        pltpu.async_copy(cf_hbm.at[f1], tab1_ref, sem1)
        pltpu.make_async_copy(cf_hbm.at[f0], tab0_ref, sem0).wait()
        compute_field(f0, tab0_ref)
        pltpu.async_copy(cf_hbm.at[f2], tab0_ref, sem0)
        pltpu.make_async_copy(cf_hbm.at[f1], tab1_ref, sem1).wait()
        compute_field(f1, tab1_ref)
        return 0
    lax.fori_loop(0, nf // 2, pair_body, 0)

    pltpu.make_async_copy(cf_hbm.at[nf - 1], tab0_ref, sem0).wait()

    pltpu.sync_copy(acc_ref, out_hbm.at[pl.ds(base, bw)])


def kernel(x, y, stds, covar_factors):
    b, nf = x.shape
    nb_cat, rank = covar_factors.shape[1], covar_factors.shape[2]
    bw = b // NW
    xf = x.reshape(-1)
    yf = y.reshape(-1)
    cf = covar_factors.reshape(nf, nb_cat * rank)
    sf = stds.reshape(-1)
    mesh = plsc.VectorSubcoreMesh(
        core_axis_name="c", subcore_axis_name="s", num_cores=NC, num_subcores=NS)
    run = pl.kernel(
        _body,
        out_type=jax.ShapeDtypeStruct((b,), jnp.float32),
        mesh=mesh,
        scratch_types=[
            pltpu.VMEM((bw * nf,), jnp.int32),
            pltpu.VMEM((bw * nf,), jnp.int32),
            pltpu.VMEM((nb_cat * rank,), jnp.float32),
            pltpu.VMEM((nb_cat * rank,), jnp.float32),
            pltpu.VMEM((nf * nb_cat,), jnp.float32),
            pltpu.VMEM((bw,), jnp.float32),
            pltpu.SemaphoreType.DMA,
            pltpu.SemaphoreType.DMA,
        ],
        compiler_params=pltpu.CompilerParams(needs_layout_passes=False),
    )
    return run(xf, yf, cf, sf)


# speedup vs baseline: 5.6546x; 1.2550x over previous
import jax
import jax.numpy as jnp
from jax import lax
from jax.experimental import pallas as pl
from jax.experimental.pallas import tpu as pltpu
from jax.experimental.pallas import tpu_sc as plsc

NC = 2
NS = 16
NW = NC * NS
L = 16


def _body(xf_hbm, yf_hbm, cf_hbm, sf_hbm, out_hbm,
          xall_ref, yall_ref, tab0_ref, tab1_ref, sall_ref, acc_ref, sem0, sem1):
    nf, tabw = cf_hbm.shape
    nb_cat = sf_hbm.shape[0] // nf
    roww = tabw // nb_cat
    rank = roww - 1
    bw = acc_ref.shape[0]
    ng = bw // L
    wid = lax.axis_index("s") * NC + lax.axis_index("c")
    base = wid * bw

    cpx = pltpu.async_copy(xf_hbm.at[pl.ds(base * nf, bw * nf)], xall_ref, sem1)
    cpy = pltpu.async_copy(yf_hbm.at[pl.ds(base * nf, bw * nf)], yall_ref, sem1)
    cps = pltpu.async_copy(sf_hbm, sall_ref, sem1)
    pltpu.async_copy(cf_hbm.at[0], tab0_ref, sem0)
    cpx.wait()
    cpy.wait()
    cps.wait()

    def zero_body(g, _):
        acc_ref[pl.ds(g * L, L)] = jnp.zeros((L,), jnp.float32)
        return 0
    lax.fori_loop(0, ng, zero_body, 0)

    iota = lax.broadcasted_iota(jnp.int32, (L,), 0)
    iota_nf = iota * nf

    def compute_field(f, tab):
        fb = f * nb_cat

        @plsc.parallel_loop(0, ng, unroll=2)
        def group_body(g):
            gb = g * (L * nf) + f
            ivx = iota_nf + gb
            xv = plsc.load_gather(xall_ref, [ivx])
            yv = plsc.load_gather(yall_ref, [ivx])
            s = plsc.load_gather(sall_ref, [xv + fb])
            xb = xv * roww
            yb = yv * roww
            na = 4
            accs = [jnp.zeros((L,), jnp.float32) for _ in range(na)]
            accs[na - 1] = jnp.where(
                xv == yv, s * s, jnp.zeros((L,), jnp.float32))
            for r in range(rank):
                xa = plsc.load_gather(tab, [xb + r])
                ya = plsc.load_gather(tab, [yb + r])
                accs[r % na] = accs[r % na] + xa * ya
            acc = (accs[0] + accs[1]) + (accs[2] + accs[3])
            sl = pl.ds(g * L, L)
            acc_ref[sl] = acc_ref[sl] + acc

    def pair_body(j, _):
        f0 = 2 * j
        f1 = f0 + 1
        f2 = jnp.minimum(f0 + 2, nf - 1)
        pltpu.async_copy(cf_hbm.at[f1], tab1_ref, sem1)
        pltpu.make_async_copy(cf_hbm.at[f0], tab0_ref, sem0).wait()
        compute_field(f0, tab0_ref)
        pltpu.async_copy(cf_hbm.at[f2], tab0_ref, sem0)
        pltpu.make_async_copy(cf_hbm.at[f1], tab1_ref, sem1).wait()
        compute_field(f1, tab1_ref)
        return 0
    lax.fori_loop(0, nf // 2, pair_body, 0)

    pltpu.make_async_copy(cf_hbm.at[nf - 1], tab0_ref, sem0).wait()

    pltpu.sync_copy(acc_ref, out_hbm.at[pl.ds(base, bw)])


def kernel(x, y, stds, covar_factors):
    b, nf = x.shape
    nb_cat, rank = covar_factors.shape[1], covar_factors.shape[2]
    bw = b // NW
    xf = x.reshape(-1)
    yf = y.reshape(-1)
    cfp = jnp.pad(covar_factors, ((0, 0), (0, 0), (0, 1)))
    cf = cfp.reshape(nf, nb_cat * (rank + 1))
    sf = stds.reshape(-1)
    mesh = plsc.VectorSubcoreMesh(
        core_axis_name="c", subcore_axis_name="s", num_cores=NC, num_subcores=NS)
    run = pl.kernel(
        _body,
        out_type=jax.ShapeDtypeStruct((b,), jnp.float32),
        mesh=mesh,
        scratch_types=[
            pltpu.VMEM((bw * nf,), jnp.int32),
            pltpu.VMEM((bw * nf,), jnp.int32),
            pltpu.VMEM((nb_cat * (rank + 1),), jnp.float32),
            pltpu.VMEM((nb_cat * (rank + 1),), jnp.float32),
            pltpu.VMEM((nf * nb_cat,), jnp.float32),
            pltpu.VMEM((bw,), jnp.float32),
            pltpu.SemaphoreType.DMA,
            pltpu.SemaphoreType.DMA,
        ],
        compiler_params=pltpu.CompilerParams(needs_layout_passes=False),
    )
    return run(xf, yf, cf, sf)


# speedup vs baseline: 5.7894x; 1.0238x over previous
import jax
import jax.numpy as jnp
from jax import lax
from jax.experimental import pallas as pl
from jax.experimental.pallas import tpu as pltpu
from jax.experimental.pallas import tpu_sc as plsc

NC = 2
NS = 16
NW = NC * NS
L = 16


def _body(x_hbm, y_hbm, cf_hbm, stds_hbm, out_hbm,
          xall_ref, yall_ref, tab0_ref, tab1_ref, sall_ref, acc_ref,
          sem0, sem1):
    nf = x_hbm.shape[1]
    nb_cat = stds_hbm.shape[1]
    rank = cf_hbm.shape[2]
    bw = acc_ref.shape[0]
    ng = bw // L
    wid = lax.axis_index("s") * NC + lax.axis_index("c")
    base = wid * bw

    cpx = pltpu.async_copy(x_hbm.at[pl.ds(base, bw)], xall_ref, sem1)
    cpy = pltpu.async_copy(y_hbm.at[pl.ds(base, bw)], yall_ref, sem1)
    cps = pltpu.async_copy(stds_hbm, sall_ref, sem1)
    pltpu.async_copy(cf_hbm.at[0], tab0_ref, sem0)
    cpx.wait()
    cpy.wait()
    cps.wait()

    def zero_body(g, _):
        acc_ref[pl.ds(g * L, L)] = jnp.zeros((L,), jnp.float32)
        return 0
    lax.fori_loop(0, ng, zero_body, 0)

    iota = lax.broadcasted_iota(jnp.int32, (L,), 0)

    def compute_field(f, tab):
        fvec = jnp.full((L,), 0, jnp.int32) + f

        @plsc.parallel_loop(0, ng, unroll=2)
        def group_body(g):
            riv = iota + g * L
            xv = plsc.load_gather(xall_ref, [riv, fvec])
            yv = plsc.load_gather(yall_ref, [riv, fvec])
            s = plsc.load_gather(sall_ref, [fvec, xv])
            na = 4
            accs = [jnp.zeros((L,), jnp.float32) for _ in range(na)]
            accs[na - 1] = jnp.where(
                xv == yv, s * s, jnp.zeros((L,), jnp.float32))
            rot = iota
            for r in range(rank):
                xa = plsc.load_gather(tab, [xv, rot])
                ya = plsc.load_gather(tab, [yv, rot])
                accs[r % na] = accs[r % na] + xa * ya
                if r + 1 < rank:
                    rot = (rot + 1) & (rank - 1)
            acc = (accs[0] + accs[1]) + (accs[2] + accs[3])
            sl = pl.ds(g * L, L)
            acc_ref[sl] = acc_ref[sl] + acc

    def pair_body(j, _):
        f0 = 2 * j
        f1 = f0 + 1
        f2 = jnp.minimum(f0 + 2, nf - 1)
        pltpu.async_copy(cf_hbm.at[f1], tab1_ref, sem1)
        pltpu.make_async_copy(cf_hbm.at[f0], tab0_ref, sem0).wait()
        compute_field(f0, tab0_ref)
        pltpu.async_copy(cf_hbm.at[f2], tab0_ref, sem0)
        pltpu.make_async_copy(cf_hbm.at[f1], tab1_ref, sem1).wait()
        compute_field(f1, tab1_ref)
        return 0
    lax.fori_loop(0, nf // 2, pair_body, 0)

    pltpu.make_async_copy(cf_hbm.at[nf - 1], tab0_ref, sem0).wait()

    pltpu.sync_copy(acc_ref, out_hbm.at[pl.ds(base, bw)])


def kernel(x, y, stds, covar_factors):
    b, nf = x.shape
    nb_cat, rank = covar_factors.shape[1], covar_factors.shape[2]
    bw = b // NW
    mesh = plsc.VectorSubcoreMesh(
        core_axis_name="c", subcore_axis_name="s", num_cores=NC, num_subcores=NS)
    run = pl.kernel(
        _body,
        out_type=jax.ShapeDtypeStruct((b,), jnp.float32),
        mesh=mesh,
        scratch_types=[
            pltpu.VMEM((bw, nf), jnp.int32),
            pltpu.VMEM((bw, nf), jnp.int32),
            pltpu.VMEM((nb_cat, rank), jnp.float32),
            pltpu.VMEM((nb_cat, rank), jnp.float32),
            pltpu.VMEM((nf, nb_cat), jnp.float32),
            pltpu.VMEM((bw,), jnp.float32),
            pltpu.SemaphoreType.DMA,
            pltpu.SemaphoreType.DMA,
        ],
        compiler_params=pltpu.CompilerParams(
            needs_layout_passes=False, use_tc_tiling_on_sc=False),
    )
    return run(x, y, covar_factors, stds)


# speedup vs baseline: 6.6060x; 1.1410x over previous
import jax
import jax.numpy as jnp
from jax import lax
from jax.experimental import pallas as pl
from jax.experimental.pallas import tpu as pltpu
from jax.experimental.pallas import tpu_sc as plsc

NC = 2
NS = 16
NW = NC * NS
L = 16


def _make_body(b, nf, nb_cat, rank):
    tabw = nb_cat * rank
    bw = b // NW
    ng = bw // L

    def _body(mi_hbm, mf_hbm, out_hbm,
              xall_ref, yall_ref, tab0_ref, tab1_ref, sall_ref, acc_ref,
              sem0, sem1):
        wid = lax.axis_index("s") * NC + lax.axis_index("c")
        base = wid * bw

        cpx = pltpu.async_copy(mi_hbm.at[pl.ds(base * nf, bw * nf)],
                               xall_ref, sem1)
        cpy = pltpu.async_copy(mi_hbm.at[pl.ds(b * nf + base * nf, bw * nf)],
                               yall_ref, sem1)
        cps = pltpu.async_copy(mf_hbm.at[pl.ds(0, nf * nb_cat)], sall_ref, sem1)
        pltpu.async_copy(mf_hbm.at[pl.ds(nf * nb_cat, tabw)], tab0_ref, sem0)
        cpx.wait()
        cpy.wait()
        cps.wait()

        def zero_body(g, _):
            acc_ref[pl.ds(g * L, L)] = jnp.zeros((L,), jnp.float32)
            return 0
        lax.fori_loop(0, ng, zero_body, 0)

        iota = lax.broadcasted_iota(jnp.int32, (L,), 0)
        iota_nf = iota * nf

        def compute_field(f, tab):
            fb = f * nb_cat

            @plsc.parallel_loop(0, ng, unroll=2)
            def group_body(g):
                ivx = iota_nf + (g * (L * nf) + f)
                xv = plsc.load_gather(xall_ref, [ivx])
                yv = plsc.load_gather(yall_ref, [ivx])
                s2 = plsc.load_gather(sall_ref, [xv + fb])
                xb = xv * rank
                yb = yv * rank
                na = 4
                accs = [jnp.zeros((L,), jnp.float32) for _ in range(na)]
                accs[na - 1] = jnp.where(
                    xv == yv, s2, jnp.zeros((L,), jnp.float32))
                rot = iota
                for r in range(rank):
                    xa = plsc.load_gather(tab, [xb + rot])
                    ya = plsc.load_gather(tab, [yb + rot])
                    accs[r % na] = accs[r % na] + xa * ya
                    if r + 1 < rank:
                        rot = (rot + 1) & (rank - 1)
                acc = (accs[0] + accs[1]) + (accs[2] + accs[3])
                sl = pl.ds(g * L, L)
                acc_ref[sl] = acc_ref[sl] + acc

        toff = nf * nb_cat

        def pair_body(j, _):
            f0 = 2 * j
            f1 = f0 + 1
            f2 = jnp.minimum(f0 + 2, nf - 1)
            pltpu.async_copy(mf_hbm.at[pl.ds(toff + f1 * tabw, tabw)],
                             tab1_ref, sem1)
            pltpu.make_async_copy(mf_hbm.at[pl.ds(toff + f0 * tabw, tabw)],
                                  tab0_ref, sem0).wait()
            compute_field(f0, tab0_ref)
            pltpu.async_copy(mf_hbm.at[pl.ds(toff + f2 * tabw, tabw)],
                             tab0_ref, sem0)
            pltpu.make_async_copy(mf_hbm.at[pl.ds(toff + f1 * tabw, tabw)],
                                  tab1_ref, sem1).wait()
            compute_field(f1, tab1_ref)
            return 0
        lax.fori_loop(0, nf // 2, pair_body, 0)

        pltpu.make_async_copy(mf_hbm.at[pl.ds(toff + (nf - 1) * tabw, tabw)],
                              tab0_ref, sem0).wait()

        pltpu.sync_copy(acc_ref, out_hbm.at[pl.ds(base, bw)])

    return _body, bw


def kernel(x, y, stds, covar_factors):
    b, nf = x.shape
    nb_cat, rank = covar_factors.shape[1], covar_factors.shape[2]
    mi = jnp.concatenate([x.reshape(-1), y.reshape(-1)])
    mf = jnp.concatenate([(stds * stds).reshape(-1),
                          covar_factors.reshape(-1)])
    body, bw = _make_body(b, nf, nb_cat, rank)
    mesh = plsc.VectorSubcoreMesh(
        core_axis_name="c", subcore_axis_name="s", num_cores=NC, num_subcores=NS)
    run = pl.kernel(
        body,
        out_type=jax.ShapeDtypeStruct((b,), jnp.float32),
        mesh=mesh,
        scratch_types=[
            pltpu.VMEM((bw * nf,), jnp.int32),
            pltpu.VMEM((bw * nf,), jnp.int32),
            pltpu.VMEM((nb_cat * rank,), jnp.float32),
            pltpu.VMEM((nb_cat * rank,), jnp.float32),
            pltpu.VMEM((nf * nb_cat,), jnp.float32),
            pltpu.VMEM((bw,), jnp.float32),
            pltpu.SemaphoreType.DMA,
            pltpu.SemaphoreType.DMA,
        ],
        compiler_params=pltpu.CompilerParams(
            needs_layout_passes=False, use_tc_tiling_on_sc=False),
    )
    return run(mi, mf)


# speedup vs baseline: 8.4627x; 1.2811x over previous
import jax
import jax.numpy as jnp
from jax import lax
from jax.experimental import pallas as pl
from jax.experimental.pallas import tpu as pltpu
from jax.experimental.pallas import tpu_sc as plsc

NC = 2
NS = 16
NW = NC * NS
L = 16


def _make_body(b, nf, nb_cat, rank):
    tabw = nb_cat * rank
    half = b // NC
    cw = half // NS
    ng = cw // L
    nu = nf * NS
    upt = nu // NS
    soff = nf * nb_cat

    def _body(xy_hbm, mf_hbm, out_hbm,
              xbuf0_ref, xbuf1_ref, tab_ref, sall_ref, part_ref, tmp_ref,
              accf_ref, shstage_ref, sem0, sem1):
        cid = lax.axis_index("c")
        sid = lax.axis_index("s")
        scbase = cid * half
        u0 = sid * upt

        cps = pltpu.async_copy(mf_hbm.at[pl.ds(0, soff)], sall_ref, sem1)
        f_0 = u0 // NS
        c_0 = u0 % NS
        pltpu.async_copy(
            xy_hbm.at[pl.ds(f_0 * b + scbase + c_0 * cw, cw)], xbuf0_ref, sem0)
        cps.wait()

        iota = lax.broadcasted_iota(jnp.int32, (L,), 0)

        def compute_unit(f, xbuf):
            fb = f * nb_cat

            @plsc.parallel_loop(0, ng, unroll=2)
            def group_body(g):
                w = xbuf[pl.ds(g * L, L)]
                xv = w & 0xFFFF
                yv = lax.shift_right_logical(w, 16)
                s2 = plsc.load_gather(sall_ref, [xv + fb])
                xb = xv * rank
                yb = yv * rank
                na = 4
                accs = [jnp.zeros((L,), jnp.float32) for _ in range(na)]
                accs[na - 1] = jnp.where(
                    xv == yv, s2, jnp.zeros((L,), jnp.float32))
                rot = iota
                for r in range(rank):
                    xa = plsc.load_gather(tab_ref, [xb + rot])
                    ya = plsc.load_gather(tab_ref, [yb + rot])
                    accs[r % na] = accs[r % na] + xa * ya
                    if r + 1 < rank:
                        rot = (rot + 1) & (rank - 1)
                acc = (accs[0] + accs[1]) + (accs[2] + accs[3])
                part_ref[pl.ds(g * L, L)] = acc

        def unit_body(k, _):
            u = u0 + k
            f = u // NS
            c = u % NS

            @pl.when(jnp.logical_or(k == 0, c == 0))
            def _():
                pltpu.sync_copy(mf_hbm.at[pl.ds(soff + f * tabw, tabw)],
                                tab_ref)

            un = jnp.minimum(u + 1, nu - 1)
            nxt = (un // NS) * b + scbase + (un % NS) * cw

            @pl.when(k % 2 == 0)
            def _():
                pltpu.async_copy(xy_hbm.at[pl.ds(nxt, cw)], xbuf1_ref, sem1)
                pltpu.make_async_copy(xy_hbm.at[pl.ds(0, cw)],
                                      xbuf0_ref, sem0).wait()
                compute_unit(f, xbuf0_ref)

            @pl.when(k % 2 == 1)
            def _():
                pltpu.async_copy(xy_hbm.at[pl.ds(nxt, cw)], xbuf0_ref, sem0)
                pltpu.make_async_copy(xy_hbm.at[pl.ds(0, cw)],
                                      xbuf1_ref, sem1).wait()
                compute_unit(f, xbuf1_ref)

            pltpu.sync_copy(part_ref, shstage_ref.at[pl.ds(u * cw, cw)])
            return 0
        lax.fori_loop(0, upt, unit_body, 0)

        pltpu.make_async_copy(xy_hbm.at[pl.ds(0, cw)], xbuf0_ref, sem0).wait()

        plsc.subcore_barrier()

        pltpu.sync_copy(shstage_ref.at[pl.ds(sid * cw, cw)], accf_ref)

        def red_body(f, _):
            pltpu.sync_copy(
                shstage_ref.at[pl.ds((f * NS + sid) * cw, cw)], tmp_ref)

            def add_body(g, _):
                sl = pl.ds(g * L, L)
                accf_ref[sl] = accf_ref[sl] + tmp_ref[sl]
                return 0
            lax.fori_loop(0, ng, add_body, 0)
            return 0
        lax.fori_loop(1, nf, red_body, 0)

        pltpu.sync_copy(accf_ref, out_hbm.at[pl.ds(scbase + sid * cw, cw)])

    return _body


def kernel(x, y, stds, covar_factors):
    b, nf = x.shape
    nb_cat, rank = covar_factors.shape[1], covar_factors.shape[2]
    xy = (x + (y << 16)).T
    eps = (stds[0, 0] * 0.0).astype(jnp.float32)
    mi = xy.reshape(-1)
    mf = jnp.concatenate([(stds * stds).reshape(-1),
                          (covar_factors + eps).reshape(-1)])
    body = _make_body(b, nf, nb_cat, rank)
    cw = b // NW
    mesh = plsc.VectorSubcoreMesh(
        core_axis_name="c", subcore_axis_name="s", num_cores=NC, num_subcores=NS)
    run = pl.kernel(
        body,
        out_type=jax.ShapeDtypeStruct((b,), jnp.float32),
        mesh=mesh,
        scratch_types=[
            pltpu.VMEM((cw,), jnp.int32),
            pltpu.VMEM((cw,), jnp.int32),
            pltpu.VMEM((nb_cat * rank,), jnp.float32),
            pltpu.VMEM((nf * nb_cat,), jnp.float32),
            pltpu.VMEM((cw,), jnp.float32),
            pltpu.VMEM((cw,), jnp.float32),
            pltpu.VMEM((cw,), jnp.float32),
            pltpu.VMEM_SHARED((nf * NS * cw,), jnp.float32),
            pltpu.SemaphoreType.DMA,
            pltpu.SemaphoreType.DMA,
        ],
        compiler_params=pltpu.CompilerParams(
            needs_layout_passes=False, use_tc_tiling_on_sc=False),
    )
    return run(mi, mf)
